# baseline (speedup 1.0000x reference)
import functools

import jax
import jax.numpy as jnp
from jax import lax
from jax.experimental import pallas as pl
from jax.experimental.pallas import tpu as pltpu
from jax.experimental.pallas import tpu_sc as plsc

N = 10000
E = 320000
B = 128

NC = 2
NS = 16
NW = NC * NS

EPT = 20480
E_PAD = NS * EPT
C = 512
K = EPT // C

NP = 10240
ROWS_PER_TILE = NP // NS
SINK = N

RBLK = 2000
G = N // RBLK


def _make_edge_agg(d2):
    mesh = plsc.VectorSubcoreMesh(core_axis_name="c", subcore_axis_name="s")

    @functools.partial(
        pl.kernel,
        out_type=jax.ShapeDtypeStruct((NC, NP, d2), jnp.float32),
        mesh=mesh,
        compiler_params=pltpu.CompilerParams(use_tc_tiling_on_sc=False),
        scratch_types=[
            pltpu.VMEM((K, C), jnp.int32),
            pltpu.VMEM((K, C), jnp.int32),
            pltpu.VMEM((C, d2), jnp.float32),
            pltpu.VMEM_SHARED((NP, d2), jnp.float32),
            pltpu.SemaphoreType.DMA,
        ],
    )
    def edge_agg(ya_hbm, yb_hbm, srcs_hbm, dsts_hbm, zeros_hbm, out_hbm,
                 src_v, dst_v, rows_v, acc_sh, sem):
        c = lax.axis_index("c")
        s = lax.axis_index("s")
        row0 = s * ROWS_PER_TILE

        for k in range(ROWS_PER_TILE // 128):
            pltpu.sync_copy(zeros_hbm, acc_sh.at[pl.ds(row0 + k * 128, 128)])

        pltpu.sync_copy(srcs_hbm.at[s], src_v)
        pltpu.sync_copy(dsts_hbm.at[s], dst_v)
        plsc.subcore_barrier()

        def body_a(j, carry):
            pltpu.async_copy(ya_hbm.at[src_v.at[j]], rows_v, sem).wait()
            pltpu.sync_copy(rows_v, acc_sh.at[dst_v.at[j]], add=True)
            return carry

        def body_b(j, carry):
            pltpu.async_copy(yb_hbm.at[src_v.at[j]], rows_v, sem).wait()
            pltpu.sync_copy(rows_v, acc_sh.at[dst_v.at[j]], add=True)
            return carry

        @pl.when(c == 0)
        def _():
            lax.fori_loop(0, K, body_a, 0)

        @pl.when(c == 1)
        def _():
            lax.fori_loop(0, K, body_b, 0)

        plsc.subcore_barrier()

        pltpu.sync_copy(acc_sh.at[pl.ds(row0, ROWS_PER_TILE)],
                        out_hbm.at[c, pl.ds(row0, ROWS_PER_TILE)])

    return edge_agg


def _matmul2_body(x_r, Wa_r, Wb_r, ya_r, yb_r):
    ya_r[...] = jnp.dot(x_r[...], Wa_r[...], preferred_element_type=jnp.float32)
    yb_r[...] = jnp.dot(x_r[...], Wb_r[...], preferred_element_type=jnp.float32)


def _matmul2(x, W):
    din = x.shape[1]
    d2 = W.shape[1] // 2
    return pl.pallas_call(
        _matmul2_body,
        grid=(G,),
        in_specs=[
            pl.BlockSpec((RBLK, din), lambda i: (i, 0)),
            pl.BlockSpec((din, d2), lambda i: (0, 0)),
            pl.BlockSpec((din, d2), lambda i: (0, 0)),
        ],
        out_specs=[
            pl.BlockSpec((RBLK, d2), lambda i: (i, 0)),
            pl.BlockSpec((RBLK, d2), lambda i: (i, 0)),
        ],
        out_shape=[
            jax.ShapeDtypeStruct((N, d2), jnp.float32),
            jax.ShapeDtypeStruct((N, d2), jnp.float32),
        ],
    )(x, W[:, :d2], W[:, d2:])


def _make_layer_body(split_next):
    def body(ya_r, yb_r, agg_r, eps_r, b1_r, W2_r, b2_r, gamma_r, beta_r,
             Wi_r, bi_r, Wo_r, bo_r, batch_r, Wna_r, Wnb_r,
             pout_r, yna_r, ynb_r, h_s, stats_s, pooled_s):
        j = pl.program_id(0)
        i = pl.program_id(1)

        @pl.when(j == 0)
        def _():
            y = jnp.concatenate([ya_r[...], yb_r[...]], axis=1)
            agg = jnp.concatenate([agg_r[0], agg_r[1]], axis=1)
            h1 = jnp.maximum(
                y * (1.0 + eps_r[0]) + agg + b1_r[...], 0.0)
            h2 = (jnp.dot(h1, W2_r[...], preferred_element_type=jnp.float32)
                  + b2_r[...])
            h_s[pl.ds(i * RBLK, RBLK), :] = h2

            @pl.when(i == 0)
            def _():
                stats_s[...] = jnp.zeros_like(stats_s)

            stats_s[0:1, :] += jnp.sum(h2, axis=0, keepdims=True)
            stats_s[1:2, :] += jnp.sum(h2 * h2, axis=0, keepdims=True)

        @pl.when(j == 1)
        def _():
            inv_n = 1.0 / N
            mean = stats_s[0:1, :] * inv_n
            ex2 = stats_s[1:2, :] * inv_n
            var = ex2 - mean * mean
            inv = lax.rsqrt(var + 1e-5)
            f = jnp.maximum(
                (h_s[pl.ds(i * RBLK, RBLK), :] - mean) * inv * gamma_r[...]
                + beta_r[...], 0.0)
            if split_next:
                yna_r[...] = jnp.dot(f, Wna_r[...],
                                     preferred_element_type=jnp.float32)
                ynb_r[...] = jnp.dot(f, Wnb_r[...],
                                     preferred_element_type=jnp.float32)
            inner = jnp.maximum(
                jnp.dot(f, Wi_r[...], preferred_element_type=jnp.float32)
                + bi_r[...], 0.0)
            bids = batch_r[0, 0, :]
            onehot = (lax.broadcasted_iota(jnp.int32, (B, RBLK), 0)
                      == bids[None, :]).astype(jnp.float32)

            @pl.when(i == 0)
            def _():
                pooled_s[...] = jnp.zeros_like(pooled_s)

            pooled_s[...] += jnp.dot(onehot, inner,
                                     preferred_element_type=jnp.float32)

            @pl.when(i == G - 1)
            def _():
                pout_r[...] = jnp.maximum(
                    jnp.dot(pooled_s[...], Wo_r[...],
                            preferred_element_type=jnp.float32) + bo_r[...],
                    0.0)

    return body


def _layer_tc(ya, yb, aggs, eps, b1, W2, b2, gamma, beta, Wi, bi, Wo, bo,
              batch3d, Wn):
    d2 = ya.shape[1]
    dout = 2 * d2
    split_next = Wn is not None

    def ph0(j, i):
        return (jnp.where(j == 0, i, 0), 0)

    def agg_map(j, i):
        return (0, jnp.where(j == 0, i, 0), 0)

    def ph1(j, i):
        return (jnp.where(j == 1, i, 0), 0)

    def full(j, i):
        return (0, 0)

    if split_next:
        dn2 = Wn.shape[1] // 2
        Wna, Wnb = Wn[:, :dn2], Wn[:, dn2:]
        yn_shape = jax.ShapeDtypeStruct((N, dn2), jnp.float32)
        yn_spec = pl.BlockSpec((RBLK, dn2), ph1)
    else:
        dn2 = 8
        Wna = Wnb = jnp.zeros((dout, dn2), jnp.float32)
        yn_shape = jax.ShapeDtypeStruct((RBLK, dn2), jnp.float32)
        yn_spec = pl.BlockSpec((RBLK, dn2), lambda j, i: (0, 0))

    pout, yna, ynb = pl.pallas_call(
        _make_layer_body(split_next),
        grid=(2, G),
        in_specs=[
            pl.BlockSpec((RBLK, d2), ph0),
            pl.BlockSpec((RBLK, d2), ph0),
            pl.BlockSpec((2, RBLK, d2), agg_map),
            pl.BlockSpec(memory_space=pltpu.SMEM),
            pl.BlockSpec((1, dout), full),
            pl.BlockSpec((dout, dout), full),
            pl.BlockSpec((1, dout), full),
            pl.BlockSpec((1, dout), full),
            pl.BlockSpec((1, dout), full),
            pl.BlockSpec((dout, dout), full),
            pl.BlockSpec((1, dout), full),
            pl.BlockSpec((dout, dout), full),
            pl.BlockSpec((1, dout), full),
            pl.BlockSpec((1, 1, RBLK), lambda j, i: (jnp.where(j == 1, i, 0),
                                                     0, 0)),
            pl.BlockSpec((dout, dn2), full),
            pl.BlockSpec((dout, dn2), full),
        ],
        out_specs=[
            pl.BlockSpec((B, dout), full),
            yn_spec,
            yn_spec,
        ],
        out_shape=[
            jax.ShapeDtypeStruct((B, dout), jnp.float32),
            yn_shape,
            yn_shape,
        ],
        scratch_shapes=[
            pltpu.VMEM((N, dout), jnp.float32),
            pltpu.VMEM((8, dout), jnp.float32),
            pltpu.VMEM((B, dout), jnp.float32),
        ],
    )(ya, yb, aggs, eps, b1, W2, b2, gamma, beta, Wi, bi, Wo, bo, batch3d,
      Wna, Wnb)
    return pout, yna, ynb


def _head_body(p1_r, p2_r, p3_r, w1a_r, w1b_r, w1c_r, b1_r, W2_r, b2_r,
               W3_r, b3_r, W4_r, b4_r, out_r):
    h = (jnp.dot(p1_r[...], w1a_r[...], preferred_element_type=jnp.float32)
         + jnp.dot(p2_r[...], w1b_r[...], preferred_element_type=jnp.float32)
         + jnp.dot(p3_r[...], w1c_r[...], preferred_element_type=jnp.float32)
         + b1_r[...])
    h = jnp.maximum(h, 0.0)
    h = jnp.tanh(
        jnp.dot(h, W2_r[...], preferred_element_type=jnp.float32) + b2_r[...])
    s = jnp.maximum(
        jnp.dot(h, W3_r[...], preferred_element_type=jnp.float32) + b3_r[...],
        0.0)
    s = jnp.dot(s, W4_r[...], preferred_element_type=jnp.float32) + b4_r[...]
    out_r[...] = 1.0 / (1.0 + jnp.exp(-s))


def _head(p1, p2, p3, w1a, w1b, w1c, b1, W2, b2, W3, b3, W4, b4):
    return pl.pallas_call(
        _head_body,
        out_shape=jax.ShapeDtypeStruct((B, 1), jnp.float32),
    )(p1, p2, p3, w1a, w1b, w1c, b1, W2, b2, W3, b3, W4, b4)


@jax.jit
def kernel(x, edge_index, batch, params):
    src = edge_index[0].astype(jnp.int32)
    dst = edge_index[1].astype(jnp.int32)
    pad = E_PAD - E
    src = jnp.concatenate([src, jnp.zeros((pad,), jnp.int32)])
    dst = jnp.concatenate([dst, jnp.full((pad,), SINK, jnp.int32)])
    srcs = src.reshape(NS, K, C)
    dsts = dst.reshape(NS, K, C)

    batch3d = batch.astype(jnp.int32).reshape(G, 1, RBLK)

    ya, yb = _matmul2(x, params['gin'][0]['W1'])
    pouts = []
    for i in range(3):
        p = params['gin'][i]
        d2 = ya.shape[1]
        zeros = jnp.zeros((128, d2), jnp.float32)
        aggs = _make_edge_agg(d2)(ya, yb, srcs, dsts, zeros)
        eps = jnp.reshape(p['eps'], (1,))
        pi = params['inner'][i]
        po = params['outer'][i]
        Wn = params['gin'][i + 1]['W1'] if i < 2 else None
        pout, ya, yb = _layer_tc(
            ya, yb, aggs, eps, p['b1'].reshape(1, -1), p['W2'],
            p['b2'].reshape(1, -1), p['gamma'].reshape(1, -1),
            p['beta'].reshape(1, -1), pi['W'], pi['b'].reshape(1, -1),
            po['W'], po['b'].reshape(1, -1), batch3d, Wn)
        pouts.append(pout)

    csW1 = params['cs_W1']
    w1a, w1b, w1c = csW1[:128], csW1[128:192], csW1[192:224]
    return _head(pouts[0], pouts[1], pouts[2],
                 w1a, w1b, w1c, params['cs_b1'].reshape(1, -1),
                 params['cs_W2'], params['cs_b2'].reshape(1, -1),
                 params['sc_W1'], params['sc_b1'].reshape(1, -1),
                 params['sc_W2'], params['sc_b2'].reshape(1, -1))


# speedup vs baseline: 1.0525x; 1.0525x over previous
import functools

import jax
import jax.numpy as jnp
from jax import lax
from jax.experimental import pallas as pl
from jax.experimental.pallas import tpu as pltpu
from jax.experimental.pallas import tpu_sc as plsc

N = 10000
E = 320000
B = 128

NC = 2
NS = 16
NW = NC * NS

EPT = 20480
E_PAD = NS * EPT
C = 512
K = EPT // C

NP = 10240
ROWS_PER_TILE = NP // NS
SINK = N

RBLK = 2000
G = N // RBLK


def _make_edge_agg(d2):
    mesh = plsc.VectorSubcoreMesh(core_axis_name="c", subcore_axis_name="s")

    @functools.partial(
        pl.kernel,
        out_type=jax.ShapeDtypeStruct((NC, NP, d2), jnp.float32),
        mesh=mesh,
        compiler_params=pltpu.CompilerParams(use_tc_tiling_on_sc=False),
        scratch_types=[
            pltpu.VMEM((K, C), jnp.int32),
            pltpu.VMEM((K, C), jnp.int32),
            pltpu.VMEM((C, d2), jnp.float32),
            pltpu.VMEM_SHARED((NP, d2), jnp.float32),
            pltpu.SemaphoreType.DMA,
        ],
    )
    def edge_agg(y2_hbm, srcs_hbm, dsts_hbm, zeros_hbm, out_hbm,
                 src_v, dst_v, rows_v, acc_sh, sem):
        c = lax.axis_index("c")
        s = lax.axis_index("s")
        wid = c * NS + s
        row0 = s * ROWS_PER_TILE

        for k in range(ROWS_PER_TILE // 128):
            pltpu.sync_copy(zeros_hbm, acc_sh.at[pl.ds(row0 + k * 128, 128)])

        pltpu.sync_copy(srcs_hbm.at[wid], src_v)
        pltpu.sync_copy(dsts_hbm.at[s], dst_v)
        plsc.subcore_barrier()

        def body(j, carry):
            pltpu.async_copy(y2_hbm.at[src_v.at[j]], rows_v, sem).wait()
            pltpu.sync_copy(rows_v, acc_sh.at[dst_v.at[j]], add=True)
            return carry

        lax.fori_loop(0, K, body, 0)
        plsc.subcore_barrier()

        pltpu.sync_copy(acc_sh.at[pl.ds(row0, ROWS_PER_TILE)],
                        out_hbm.at[c, pl.ds(row0, ROWS_PER_TILE)])

    return edge_agg


def _matmul2_body(x_r, Wa_r, Wb_r, y2_r):
    y2_r[0] = jnp.dot(x_r[...], Wa_r[...], preferred_element_type=jnp.float32)
    y2_r[1] = jnp.dot(x_r[...], Wb_r[...], preferred_element_type=jnp.float32)


def _matmul2(x, W):
    din = x.shape[1]
    d2 = W.shape[1] // 2
    return pl.pallas_call(
        _matmul2_body,
        grid=(G,),
        in_specs=[
            pl.BlockSpec((RBLK, din), lambda i: (i, 0)),
            pl.BlockSpec((din, d2), lambda i: (0, 0)),
            pl.BlockSpec((din, d2), lambda i: (0, 0)),
        ],
        out_specs=pl.BlockSpec((2, RBLK, d2), lambda i: (0, i, 0)),
        out_shape=jax.ShapeDtypeStruct((2, N, d2), jnp.float32),
    )(x, W[:, :d2], W[:, d2:])


def _make_layer_body(split_next):
    def body(y_r, agg_r, eps_r, b1_r, W2_r, b2_r, gamma_r, beta_r,
             Wi_r, bi_r, Wo_r, bo_r, batch_r, Wna_r, Wnb_r,
             pout_r, ynext_r, h_s, stats_s, pooled_s):
        j = pl.program_id(0)
        i = pl.program_id(1)

        @pl.when(j == 0)
        def _():
            y = jnp.concatenate([y_r[0], y_r[1]], axis=1)
            agg = jnp.concatenate([agg_r[0], agg_r[1]], axis=1)
            h1 = jnp.maximum(
                y * (1.0 + eps_r[0]) + agg + b1_r[...], 0.0)
            h2 = (jnp.dot(h1, W2_r[...], preferred_element_type=jnp.float32)
                  + b2_r[...])
            h_s[pl.ds(i * RBLK, RBLK), :] = h2

            @pl.when(i == 0)
            def _():
                stats_s[...] = jnp.zeros_like(stats_s)

            stats_s[0:1, :] += jnp.sum(h2, axis=0, keepdims=True)
            stats_s[1:2, :] += jnp.sum(h2 * h2, axis=0, keepdims=True)

        @pl.when(j == 1)
        def _():
            inv_n = 1.0 / N
            mean = stats_s[0:1, :] * inv_n
            ex2 = stats_s[1:2, :] * inv_n
            var = ex2 - mean * mean
            inv = lax.rsqrt(var + 1e-5)
            f = jnp.maximum(
                (h_s[pl.ds(i * RBLK, RBLK), :] - mean) * inv * gamma_r[...]
                + beta_r[...], 0.0)
            if split_next:
                ynext_r[0] = jnp.dot(f, Wna_r[...],
                                     preferred_element_type=jnp.float32)
                ynext_r[1] = jnp.dot(f, Wnb_r[...],
                                     preferred_element_type=jnp.float32)
            inner = jnp.maximum(
                jnp.dot(f, Wi_r[...], preferred_element_type=jnp.float32)
                + bi_r[...], 0.0)
            bids = batch_r[0, 0, :]
            onehot = (lax.broadcasted_iota(jnp.int32, (B, RBLK), 0)
                      == bids[None, :]).astype(jnp.float32)

            @pl.when(i == 0)
            def _():
                pooled_s[...] = jnp.zeros_like(pooled_s)

            pooled_s[...] += jnp.dot(onehot, inner,
                                     preferred_element_type=jnp.float32)

            @pl.when(i == G - 1)
            def _():
                pout_r[...] = jnp.maximum(
                    jnp.dot(pooled_s[...], Wo_r[...],
                            preferred_element_type=jnp.float32) + bo_r[...],
                    0.0)

    return body


def _layer_tc(y2, aggs, eps, b1, W2, b2, gamma, beta, Wi, bi, Wo, bo,
              batch3d, Wn):
    d2 = y2.shape[2]
    dout = 2 * d2
    split_next = Wn is not None

    def ph0(j, i):
        return (0, jnp.where(j == 0, i, 0), 0)

    def ph1(j, i):
        return (0, jnp.where(j == 1, i, 0), 0)

    def full(j, i):
        return (0, 0)

    if split_next:
        dn2 = Wn.shape[1] // 2
        Wna, Wnb = Wn[:, :dn2], Wn[:, dn2:]
        ynext_shape = jax.ShapeDtypeStruct((2, N, dn2), jnp.float32)
        ynext_spec = pl.BlockSpec((2, RBLK, dn2), ph1)
    else:
        dn2 = 8
        Wna = Wnb = jnp.zeros((dout, dn2), jnp.float32)
        ynext_shape = jax.ShapeDtypeStruct((2, RBLK, dn2), jnp.float32)
        ynext_spec = pl.BlockSpec((2, RBLK, dn2), lambda j, i: (0, 0, 0))

    pout, ynext = pl.pallas_call(
        _make_layer_body(split_next),
        grid=(2, G),
        in_specs=[
            pl.BlockSpec((2, RBLK, d2), ph0),
            pl.BlockSpec((2, RBLK, d2), ph0),
            pl.BlockSpec(memory_space=pltpu.SMEM),
            pl.BlockSpec((1, dout), full),
            pl.BlockSpec((dout, dout), full),
            pl.BlockSpec((1, dout), full),
            pl.BlockSpec((1, dout), full),
            pl.BlockSpec((1, dout), full),
            pl.BlockSpec((dout, dout), full),
            pl.BlockSpec((1, dout), full),
            pl.BlockSpec((dout, dout), full),
            pl.BlockSpec((1, dout), full),
            pl.BlockSpec((1, 1, RBLK), lambda j, i: (jnp.where(j == 1, i, 0),
                                                     0, 0)),
            pl.BlockSpec((dout, dn2), full),
            pl.BlockSpec((dout, dn2), full),
        ],
        out_specs=[
            pl.BlockSpec((B, dout), full),
            ynext_spec,
        ],
        out_shape=[
            jax.ShapeDtypeStruct((B, dout), jnp.float32),
            ynext_shape,
        ],
        scratch_shapes=[
            pltpu.VMEM((N, dout), jnp.float32),
            pltpu.VMEM((8, dout), jnp.float32),
            pltpu.VMEM((B, dout), jnp.float32),
        ],
    )(y2, aggs, eps, b1, W2, b2, gamma, beta, Wi, bi, Wo, bo, batch3d,
      Wna, Wnb)
    return pout, ynext


def _head_body(p1_r, p2_r, p3_r, w1a_r, w1b_r, w1c_r, b1_r, W2_r, b2_r,
               W3_r, b3_r, W4_r, b4_r, out_r):
    h = (jnp.dot(p1_r[...], w1a_r[...], preferred_element_type=jnp.float32)
         + jnp.dot(p2_r[...], w1b_r[...], preferred_element_type=jnp.float32)
         + jnp.dot(p3_r[...], w1c_r[...], preferred_element_type=jnp.float32)
         + b1_r[...])
    h = jnp.maximum(h, 0.0)
    h = jnp.tanh(
        jnp.dot(h, W2_r[...], preferred_element_type=jnp.float32) + b2_r[...])
    s = jnp.maximum(
        jnp.dot(h, W3_r[...], preferred_element_type=jnp.float32) + b3_r[...],
        0.0)
    s = jnp.dot(s, W4_r[...], preferred_element_type=jnp.float32) + b4_r[...]
    out_r[...] = 1.0 / (1.0 + jnp.exp(-s))


def _head(p1, p2, p3, w1a, w1b, w1c, b1, W2, b2, W3, b3, W4, b4):
    return pl.pallas_call(
        _head_body,
        out_shape=jax.ShapeDtypeStruct((B, 1), jnp.float32),
    )(p1, p2, p3, w1a, w1b, w1c, b1, W2, b2, W3, b3, W4, b4)


@jax.jit
def kernel(x, edge_index, batch, params):
    src = edge_index[0].astype(jnp.int32)
    dst = edge_index[1].astype(jnp.int32)
    pad = E_PAD - E
    src = jnp.concatenate([src, jnp.zeros((pad,), jnp.int32)])
    dst = jnp.concatenate([dst, jnp.full((pad,), SINK, jnp.int32)])
    srcs = jnp.stack([src, src + N]).reshape(NC, NS, K, C).reshape(NW, K, C)
    dsts = dst.reshape(NS, K, C)

    batch3d = batch.astype(jnp.int32).reshape(G, 1, RBLK)

    y2 = _matmul2(x, params['gin'][0]['W1'])
    pouts = []
    for i in range(3):
        p = params['gin'][i]
        d2 = y2.shape[2]
        zeros = jnp.zeros((128, d2), jnp.float32)
        aggs = _make_edge_agg(d2)(y2.reshape(2 * N, d2), srcs, dsts, zeros)
        eps = jnp.reshape(p['eps'], (1,))
        pi = params['inner'][i]
        po = params['outer'][i]
        Wn = params['gin'][i + 1]['W1'] if i < 2 else None
        pout, y2 = _layer_tc(
            y2, aggs, eps, p['b1'].reshape(1, -1), p['W2'],
            p['b2'].reshape(1, -1), p['gamma'].reshape(1, -1),
            p['beta'].reshape(1, -1), pi['W'], pi['b'].reshape(1, -1),
            po['W'], po['b'].reshape(1, -1), batch3d, Wn)
        pouts.append(pout)

    csW1 = params['cs_W1']
    w1a, w1b, w1c = csW1[:128], csW1[128:192], csW1[192:224]
    return _head(pouts[0], pouts[1], pouts[2],
                 w1a, w1b, w1c, params['cs_b1'].reshape(1, -1),
                 params['cs_W2'], params['cs_b2'].reshape(1, -1),
                 params['sc_W1'], params['sc_b1'].reshape(1, -1),
                 params['sc_W2'], params['sc_b2'].reshape(1, -1))
